# baseline (speedup 1.0000x reference)
import jax
import jax.numpy as jnp
from jax import lax
from jax.experimental import pallas as pl
from jax.experimental.pallas import tpu as pltpu
from jax.experimental.pallas import tpu_sc as plsc

BATCH = 4
SEQ_LEN = 8192
EMBED = 1024

S_BLK = 512


def _add_body(x_ref, pos_ref, sc_ref, out_ref):
    del sc_ref
    out_ref[...] = x_ref[...] + pos_ref[...]


def _tc_part(x, pos_table, sc_out, seq_start):
    blk0 = seq_start // S_BLK
    grid = ((SEQ_LEN - seq_start) // S_BLK, BATCH)
    return pl.pallas_call(
        _add_body,
        grid=grid,
        in_specs=[
            pl.BlockSpec((1, S_BLK, EMBED), lambda s, b: (b, s + blk0, 0)),
            pl.BlockSpec((S_BLK, EMBED), lambda s, b: (s + blk0, 0)),
            pl.BlockSpec(memory_space=pltpu.MemorySpace.HBM),
        ],
        out_specs=pl.BlockSpec((1, S_BLK, EMBED), lambda s, b: (b, s + blk0, 0)),
        out_shape=jax.ShapeDtypeStruct((BATCH, SEQ_LEN, EMBED), jnp.float32),
        input_output_aliases={2: 0},
        compiler_params=pltpu.CompilerParams(
            dimension_semantics=("parallel", "arbitrary"),
        ),
    )(x, pos_table, sc_out)


NC, NS, L = 2, 16, 16
NW = NC * NS
SEQ_SC = 2048
R = 16
NBUF = 3


def _sc_body(x_hbm, pos_hbm, out_hbm, xbufs, posbuf, gsems, ssems, psem):
    w = lax.axis_index("s") * NC + lax.axis_index("c")
    rows_per_w = SEQ_SC // NW
    chunks = rows_per_w // R
    seq0 = w * rows_per_w

    @pl.loop(0, chunks)
    def chunk_body(c):
        prow = seq0 + c * R
        pltpu.async_copy(pos_hbm.at[pl.ds(prow, R), :], posbuf, psem).wait()

        def x_rows(b):
            return (pl.ds(b * SEQ_LEN + prow, R), slice(None))


        h_g = {0: pltpu.async_copy(x_hbm.at[x_rows(0)], xbufs[0], gsems[0])}
        h_s = {}
        for b in range(BATCH):
            if b + 1 < BATCH:
                if b + 1 - NBUF >= 0:
                    h_s.pop(b + 1 - NBUF).wait()
                h_g[b + 1] = pltpu.async_copy(
                    x_hbm.at[x_rows(b + 1)], xbufs[(b + 1) % NBUF],
                    gsems[(b + 1) % NBUF])
            h_g.pop(b).wait()

            xbuf = xbufs[b % NBUF]

            def add_one(j, xbuf=xbuf):
                sl = pl.ds(j * L, L)
                for r in range(R):
                    plsc.addupdate(xbuf.at[r, sl], posbuf[r, sl])

            plsc.parallel_loop(0, EMBED // L, 1, unroll=2)(add_one)
            h_s[b] = pltpu.async_copy(xbuf, out_hbm.at[x_rows(b)],
                                      ssems[b % NBUF])
        for b in sorted(h_s):
            h_s[b].wait()


def _sc_part(x, pos_table):
    xf = x.reshape(BATCH * SEQ_LEN, EMBED)
    out = pl.kernel(
        _sc_body,
        out_type=jax.ShapeDtypeStruct((BATCH * SEQ_LEN, EMBED), jnp.float32),
        mesh=plsc.VectorSubcoreMesh(core_axis_name="c", subcore_axis_name="s"),
        scratch_types=[
            [pltpu.VMEM((R, EMBED), jnp.float32) for _ in range(NBUF)],
            pltpu.VMEM((R, EMBED), jnp.float32),
            [pltpu.SemaphoreType.DMA for _ in range(NBUF)],
            [pltpu.SemaphoreType.DMA for _ in range(NBUF)],
            pltpu.SemaphoreType.DMA,
        ],
        compiler_params=pltpu.CompilerParams(use_tc_tiling_on_sc=True),
    )(xf, pos_table)
    return out.reshape(BATCH, SEQ_LEN, EMBED)


def kernel(x, pos_table):
    sc_out = _sc_part(x, pos_table)
    return _tc_part(x, pos_table, sc_out, SEQ_SC)


# speedup vs baseline: 1.1314x; 1.1314x over previous
import jax
import jax.numpy as jnp
from jax import lax
from jax.experimental import pallas as pl
from jax.experimental.pallas import tpu as pltpu
from jax.experimental.pallas import tpu_sc as plsc

BATCH = 4
SEQ_LEN = 8192
EMBED = 1024

S_BLK = 1024


def _add_body(x_ref, pos_ref, sc_ref, out_ref):
    del sc_ref
    out_ref[...] = x_ref[...] + pos_ref[...]


def _tc_part(x, pos_table, sc_out, seq_start):
    blk0 = seq_start // S_BLK
    grid = ((SEQ_LEN - seq_start) // S_BLK, BATCH)
    return pl.pallas_call(
        _add_body,
        grid=grid,
        in_specs=[
            pl.BlockSpec((1, S_BLK, EMBED), lambda s, b: (b, s + blk0, 0)),
            pl.BlockSpec((S_BLK, EMBED), lambda s, b: (s + blk0, 0)),
            pl.BlockSpec(memory_space=pltpu.MemorySpace.HBM),
        ],
        out_specs=pl.BlockSpec((1, S_BLK, EMBED), lambda s, b: (b, s + blk0, 0)),
        out_shape=jax.ShapeDtypeStruct((BATCH, SEQ_LEN, EMBED), jnp.float32),
        input_output_aliases={2: 0},
        compiler_params=pltpu.CompilerParams(
            dimension_semantics=("parallel", "arbitrary"),
        ),
    )(x, pos_table, sc_out)


NC, NS, L = 2, 16, 16
NW = NC * NS
SEQ_SC = 1024
R = 16
NBUF = 3


def _sc_body(x_hbm, pos_hbm, out_hbm, xbufs, posbuf, gsems, ssems, psem):
    w = lax.axis_index("s") * NC + lax.axis_index("c")
    rows_per_w = SEQ_SC // NW
    chunks = rows_per_w // R
    seq0 = w * rows_per_w

    @pl.loop(0, chunks)
    def chunk_body(c):
        prow = seq0 + c * R
        pltpu.async_copy(pos_hbm.at[pl.ds(prow, R), :], posbuf, psem).wait()

        def x_rows(b):
            return (pl.ds(b * SEQ_LEN + prow, R), slice(None))


        h_g = {0: pltpu.async_copy(x_hbm.at[x_rows(0)], xbufs[0], gsems[0])}
        h_s = {}
        for b in range(BATCH):
            if b + 1 < BATCH:
                if b + 1 - NBUF >= 0:
                    h_s.pop(b + 1 - NBUF).wait()
                h_g[b + 1] = pltpu.async_copy(
                    x_hbm.at[x_rows(b + 1)], xbufs[(b + 1) % NBUF],
                    gsems[(b + 1) % NBUF])
            h_g.pop(b).wait()

            xbuf = xbufs[b % NBUF]

            def add_one(j, xbuf=xbuf):
                sl = pl.ds(j * L, L)
                for r in range(R):
                    plsc.addupdate(xbuf.at[r, sl], posbuf[r, sl])

            plsc.parallel_loop(0, EMBED // L, 1, unroll=2)(add_one)
            h_s[b] = pltpu.async_copy(xbuf, out_hbm.at[x_rows(b)],
                                      ssems[b % NBUF])
        for b in sorted(h_s):
            h_s[b].wait()


def _sc_part(x, pos_table):
    xf = x.reshape(BATCH * SEQ_LEN, EMBED)
    out = pl.kernel(
        _sc_body,
        out_type=jax.ShapeDtypeStruct((BATCH * SEQ_LEN, EMBED), jnp.float32),
        mesh=plsc.VectorSubcoreMesh(core_axis_name="c", subcore_axis_name="s"),
        scratch_types=[
            [pltpu.VMEM((R, EMBED), jnp.float32) for _ in range(NBUF)],
            pltpu.VMEM((R, EMBED), jnp.float32),
            [pltpu.SemaphoreType.DMA for _ in range(NBUF)],
            [pltpu.SemaphoreType.DMA for _ in range(NBUF)],
            pltpu.SemaphoreType.DMA,
        ],
        compiler_params=pltpu.CompilerParams(use_tc_tiling_on_sc=True),
    )(xf, pos_table)
    return out.reshape(BATCH, SEQ_LEN, EMBED)


def kernel(x, pos_table):
    sc_out = _sc_part(x, pos_table)
    return _tc_part(x, pos_table, sc_out, SEQ_SC)


# speedup vs baseline: 1.2328x; 1.0897x over previous
import jax
import jax.numpy as jnp
from jax import lax
from jax.experimental import pallas as pl
from jax.experimental.pallas import tpu as pltpu
from jax.experimental.pallas import tpu_sc as plsc

BATCH = 4
SEQ_LEN = 8192
EMBED = 1024

S_BLK = 1024


def _add_body(x_ref, pos_ref, sc_ref, out_ref):
    del sc_ref
    out_ref[...] = x_ref[...] + pos_ref[...]


def _tc_part(x, pos_table, sc_out, seq_start):
    blk0 = seq_start // S_BLK
    grid = ((SEQ_LEN - seq_start) // S_BLK, BATCH)
    return pl.pallas_call(
        _add_body,
        grid=grid,
        in_specs=[
            pl.BlockSpec((1, S_BLK, EMBED), lambda s, b: (b, s + blk0, 0)),
            pl.BlockSpec((S_BLK, EMBED), lambda s, b: (s + blk0, 0)),
            pl.BlockSpec(memory_space=pltpu.MemorySpace.HBM),
        ],
        out_specs=pl.BlockSpec((1, S_BLK, EMBED), lambda s, b: (b, s + blk0, 0)),
        out_shape=jax.ShapeDtypeStruct((BATCH, SEQ_LEN, EMBED), jnp.float32),
        input_output_aliases={2: 0},
        compiler_params=pltpu.CompilerParams(
            dimension_semantics=("parallel", "arbitrary"),
        ),
    )(x, pos_table, sc_out)


NC, NS, L = 2, 16, 16
NW = NC * NS
SEQ_SC = 512
R = 16
NBUF = 3


def _sc_body(x_hbm, pos_hbm, out_hbm, xbufs, posbuf, gsems, ssems, psem):
    w = lax.axis_index("s") * NC + lax.axis_index("c")
    rows_per_w = SEQ_SC // NW
    chunks = rows_per_w // R
    seq0 = w * rows_per_w

    @pl.loop(0, chunks)
    def chunk_body(c):
        prow = seq0 + c * R
        pltpu.async_copy(pos_hbm.at[pl.ds(prow, R), :], posbuf, psem).wait()

        def x_rows(b):
            return (pl.ds(b * SEQ_LEN + prow, R), slice(None))


        h_g = {0: pltpu.async_copy(x_hbm.at[x_rows(0)], xbufs[0], gsems[0])}
        h_s = {}
        for b in range(BATCH):
            if b + 1 < BATCH:
                if b + 1 - NBUF >= 0:
                    h_s.pop(b + 1 - NBUF).wait()
                h_g[b + 1] = pltpu.async_copy(
                    x_hbm.at[x_rows(b + 1)], xbufs[(b + 1) % NBUF],
                    gsems[(b + 1) % NBUF])
            h_g.pop(b).wait()

            xbuf = xbufs[b % NBUF]

            def add_one(j, xbuf=xbuf):
                sl = pl.ds(j * L, L)
                for r in range(R):
                    plsc.addupdate(xbuf.at[r, sl], posbuf[r, sl])

            plsc.parallel_loop(0, EMBED // L, 1, unroll=2)(add_one)
            h_s[b] = pltpu.async_copy(xbuf, out_hbm.at[x_rows(b)],
                                      ssems[b % NBUF])
        for b in sorted(h_s):
            h_s[b].wait()


def _sc_part(x, pos_table):
    xf = x.reshape(BATCH * SEQ_LEN, EMBED)
    out = pl.kernel(
        _sc_body,
        out_type=jax.ShapeDtypeStruct((BATCH * SEQ_LEN, EMBED), jnp.float32),
        mesh=plsc.VectorSubcoreMesh(core_axis_name="c", subcore_axis_name="s"),
        scratch_types=[
            [pltpu.VMEM((R, EMBED), jnp.float32) for _ in range(NBUF)],
            pltpu.VMEM((R, EMBED), jnp.float32),
            [pltpu.SemaphoreType.DMA for _ in range(NBUF)],
            [pltpu.SemaphoreType.DMA for _ in range(NBUF)],
            pltpu.SemaphoreType.DMA,
        ],
        compiler_params=pltpu.CompilerParams(use_tc_tiling_on_sc=True),
    )(xf, pos_table)
    return out.reshape(BATCH, SEQ_LEN, EMBED)


def kernel(x, pos_table):
    sc_out = _sc_part(x, pos_table)
    return _tc_part(x, pos_table, sc_out, SEQ_SC)
